# initial kernel scaffold (unmeasured)
import jax
import jax.numpy as jnp
from jax import lax
from jax.experimental import pallas as pl
from jax.experimental.pallas import tpu as pltpu

N_DEV = 4
SQ = 512
SKV = 2048
HQ = 8
DH = 128
DM = 1024
SCALE = 0.08838834764831843
STAT = DH and 4
ROWS = SQ + 2 * 4


def kernel(x, Wq, Wo, K_ext, V_ext):
    def body(x_ref, wq_ref, wo_ref, k_ref, v_ref, out_ref,
             comm_ref, send_sems, recv_sems):
        my = lax.axis_index("i")

        barrier = pltpu.get_barrier_semaphore()
        for d in range(1, N_DEV):
            pl.semaphore_signal(
                barrier, inc=1,
                device_id=((my + d) % N_DEV,),
                device_id_type=pl.DeviceIdType.MESH,
            )
        pl.semaphore_wait(barrier, N_DEV - 1)

        q = jnp.dot(x_ref[0], wq_ref[...], preferred_element_type=jnp.float32)

        packs = []
        for h in range(HQ):
            qh = q[:, h * DH:(h + 1) * DH]
            kh = k_ref[0, :, h, :]
            vh = v_ref[0, :, h, :]
            s = lax.dot_general(
                qh, kh, (((1,), (1,)), ((), ())),
                preferred_element_type=jnp.float32,
            ) * SCALE
            mh = jnp.max(s, axis=1, keepdims=True)
            p = jnp.exp(s - mh)
            lh = jnp.sum(p, axis=1, keepdims=True)
            oh = jnp.dot(p, vh, preferred_element_type=jnp.float32)
            packs.append(jnp.concatenate(
                [oh, mh.reshape(4, DH), lh.reshape(4, DH)], axis=0))
        comm_ref[0] = jnp.stack(packs, axis=0)

        sends = []
        for d in range(1, N_DEV):
            rdma = pltpu.make_async_remote_copy(
                src_ref=comm_ref.at[0],
                dst_ref=comm_ref.at[N_DEV - d],
                send_sem=send_sems.at[d - 1],
                recv_sem=recv_sems.at[N_DEV - d],
                device_id=((my + d) % N_DEV,),
                device_id_type=pl.DeviceIdType.MESH,
            )
            rdma.start()
            sends.append(rdma)

        for r in range(1, N_DEV):
            recv = pltpu.make_async_remote_copy(
                src_ref=comm_ref.at[r],
                dst_ref=comm_ref.at[r],
                send_sem=send_sems.at[0],
                recv_sem=recv_sems.at[r],
                device_id=((my + r) % N_DEV,),
                device_id_type=pl.DeviceIdType.MESH,
            )
            recv.wait_recv()

        ms, ls, os_ = [], [], []
        for r in range(N_DEV):
            ms.append(comm_ref[r, :, SQ:SQ + 4, :].reshape(HQ, SQ))
            ls.append(comm_ref[r, :, SQ + 4:SQ + 8, :].reshape(HQ, SQ))
            os_.append(comm_ref[r, :, :SQ, :])
        m_tot = jnp.maximum(jnp.maximum(ms[0], ms[1]),
                            jnp.maximum(ms[2], ms[3]))
        l_tot = 0.0
        o_tot = 0.0
        for r in range(N_DEV):
            a = jnp.exp(ms[r] - m_tot)
            l_tot = l_tot + ls[r] * a
            o_tot = o_tot + os_[r] * a[:, :, None]
        o_fin = o_tot / l_tot[:, :, None]

        attn = jnp.concatenate([o_fin[h] for h in range(HQ)], axis=1)
        out_ref[0] = jnp.dot(attn, wo_ref[...],
                             preferred_element_type=jnp.float32)

        for rdma in sends:
            rdma.wait_send()

    return pl.pallas_call(
        body,
        out_shape=jax.ShapeDtypeStruct((1, SQ, DM), jnp.float32),
        in_specs=[pl.BlockSpec(memory_space=pltpu.VMEM)] * 5,
        out_specs=pl.BlockSpec(memory_space=pltpu.VMEM),
        scratch_shapes=[
            pltpu.VMEM((N_DEV, HQ, ROWS, DH), jnp.float32),
            pltpu.SemaphoreType.DMA((N_DEV - 1,)),
            pltpu.SemaphoreType.DMA((N_DEV,)),
        ],
        compiler_params=pltpu.CompilerParams(collective_id=0),
    )(x, Wq, Wo, K_ext, V_ext)


# baseline (device time: 110336 ns/iter reference)
import jax
import jax.numpy as jnp
from jax import lax
from jax.experimental import pallas as pl
from jax.experimental.pallas import tpu as pltpu

N_DEV = 4
SQ = 512
SKV = 2048
HQ = 8
DH = 128
DM = 1024
SCALE = 0.08838834764831843


def kernel(x, Wq, Wo, K_ext, V_ext):
    def body(x_ref, wq_ref, wo_ref, k_ref, v_ref, out_ref,
             comm_o, comm_s, q_ref, attn_ref, kh_ref, vh_ref,
             kv_sems, send_o_sems, send_s_sems, recv_o_sems, recv_s_sems):
        my = lax.axis_index("i")

        barrier = pltpu.get_barrier_semaphore()
        for d in range(1, N_DEV):
            pl.semaphore_signal(
                barrier, inc=1,
                device_id=((my + d) % N_DEV,),
                device_id_type=pl.DeviceIdType.MESH,
            )
        pl.semaphore_wait(barrier, N_DEV - 1)

        q_ref[...] = jnp.dot(x_ref[0], wq_ref[...],
                             preferred_element_type=jnp.float32)

        for h in range(HQ):
            cp_k = pltpu.make_async_copy(
                k_ref.at[0, :, h, :], kh_ref, kv_sems.at[0])
            cp_v = pltpu.make_async_copy(
                v_ref.at[0, :, h, :], vh_ref, kv_sems.at[1])
            cp_k.start()
            cp_v.start()
            cp_k.wait()
            cp_v.wait()
            qh = q_ref[:, h * DH:(h + 1) * DH]
            s = lax.dot_general(
                qh, kh_ref[...], (((1,), (1,)), ((), ())),
                preferred_element_type=jnp.float32,
            ) * SCALE
            mh = jnp.max(s, axis=1, keepdims=True)
            p = jnp.exp(s - mh)
            comm_o[0, h] = jnp.dot(p, vh_ref[...],
                                   preferred_element_type=jnp.float32)
            comm_s[0, :, h:h + 1] = mh
            comm_s[0, :, HQ + h:HQ + h + 1] = jnp.sum(
                p, axis=1, keepdims=True)

        sends = []
        for d in range(1, N_DEV):
            peer = (my + d) % N_DEV
            for src, ssem, rsem in (
                (comm_o, send_o_sems, recv_o_sems),
                (comm_s, send_s_sems, recv_s_sems),
            ):
                rdma = pltpu.make_async_remote_copy(
                    src_ref=src.at[0],
                    dst_ref=src.at[N_DEV - d],
                    send_sem=ssem.at[d - 1],
                    recv_sem=rsem.at[N_DEV - d],
                    device_id=(peer,),
                    device_id_type=pl.DeviceIdType.MESH,
                )
                rdma.start()
                sends.append(rdma)

        for r in range(1, N_DEV):
            for buf, rsem, ssem in (
                (comm_o, recv_o_sems, send_o_sems),
                (comm_s, recv_s_sems, send_s_sems),
            ):
                recv = pltpu.make_async_remote_copy(
                    src_ref=buf.at[r],
                    dst_ref=buf.at[r],
                    send_sem=ssem.at[0],
                    recv_sem=rsem.at[r],
                    device_id=((my + r) % N_DEV,),
                    device_id_type=pl.DeviceIdType.MESH,
                )
                recv.wait_recv()

        for h in range(HQ):
            ms = [comm_s[r, :, h:h + 1] for r in range(N_DEV)]
            m_tot = jnp.maximum(jnp.maximum(ms[0], ms[1]),
                                jnp.maximum(ms[2], ms[3]))
            l_tot = 0.0
            o_tot = 0.0
            for r in range(N_DEV):
                a = jnp.exp(ms[r] - m_tot)
                l_tot = l_tot + comm_s[r, :, HQ + h:HQ + h + 1] * a
                o_tot = o_tot + comm_o[r, h] * a
            attn_ref[:, h * DH:(h + 1) * DH] = o_tot / l_tot

        out_ref[0] = jnp.dot(attn_ref[...], wo_ref[...],
                             preferred_element_type=jnp.float32)

        for rdma in sends:
            rdma.wait_send()

    return pl.pallas_call(
        body,
        out_shape=jax.ShapeDtypeStruct((1, SQ, DM), jnp.float32),
        in_specs=[
            pl.BlockSpec(memory_space=pltpu.VMEM),
            pl.BlockSpec(memory_space=pltpu.VMEM),
            pl.BlockSpec(memory_space=pltpu.VMEM),
            pl.BlockSpec(memory_space=pl.ANY),
            pl.BlockSpec(memory_space=pl.ANY),
        ],
        out_specs=pl.BlockSpec(memory_space=pltpu.VMEM),
        scratch_shapes=[
            pltpu.VMEM((N_DEV, HQ, SQ, DH), jnp.float32),
            pltpu.VMEM((N_DEV, SQ, 2 * HQ), jnp.float32),
            pltpu.VMEM((SQ, DM), jnp.float32),
            pltpu.VMEM((SQ, DM), jnp.float32),
            pltpu.VMEM((SKV, DH), jnp.float32),
            pltpu.VMEM((SKV, DH), jnp.float32),
            pltpu.SemaphoreType.DMA((2,)),
            pltpu.SemaphoreType.DMA((N_DEV - 1,)),
            pltpu.SemaphoreType.DMA((N_DEV - 1,)),
            pltpu.SemaphoreType.DMA((N_DEV,)),
            pltpu.SemaphoreType.DMA((N_DEV,)),
        ],
        compiler_params=pltpu.CompilerParams(
            collective_id=0,
            vmem_limit_bytes=100 * 1024 * 1024,
        ),
    )(x, Wq, Wo, K_ext, V_ext)


# device time: 61473 ns/iter; 1.7949x vs baseline; 1.7949x over previous
import jax
import jax.numpy as jnp
from jax import lax
from jax.experimental import pallas as pl
from jax.experimental.pallas import tpu as pltpu

N_DEV = 4
SQ = 512
SQC = SQ // N_DEV
SKV = 2048
HQ = 8
DH = 128
DM = 1024
SCALE = 0.08838834764831843
NHG = HQ // 2


def kernel(x, Wq, Wo, K_ext, V_ext):
    def body(x_ref, wq_ref, wo_ref, k_ref, v_ref, out_ref,
             loc_o, loc_s, rs_o, rs_s, my_o, my_s,
             q_ref, attn_c, rows_ref, ag_out, kh_ref, vh_ref,
             kv_sems, self_sems, rs_send_o, rs_recv_o,
             rs_send_s, rs_recv_s, ag_send, ag_recv):
        my = lax.axis_index("i")

        barrier = pltpu.get_barrier_semaphore()
        for d in range(1, N_DEV):
            pl.semaphore_signal(
                barrier, inc=1,
                device_id=((my + d) % N_DEV,),
                device_id_type=pl.DeviceIdType.MESH,
            )
        pl.semaphore_wait(barrier, N_DEV - 1)

        def kv_fetch(h, slot):
            cp_k = pltpu.make_async_copy(
                k_ref.at[0, :, h, :], kh_ref.at[slot], kv_sems.at[slot, 0])
            cp_v = pltpu.make_async_copy(
                v_ref.at[0, :, h, :], vh_ref.at[slot], kv_sems.at[slot, 1])
            cp_k.start()
            cp_v.start()
            return cp_k, cp_v

        kv_fetch(0, 0)

        q_ref[...] = jnp.dot(x_ref[0], wq_ref[...],
                             preferred_element_type=jnp.float32)

        sends = []
        for h in range(HQ):
            slot = h % 2
            pltpu.make_async_copy(
                k_ref.at[0, :, h, :], kh_ref.at[slot],
                kv_sems.at[slot, 0]).wait()
            pltpu.make_async_copy(
                v_ref.at[0, :, h, :], vh_ref.at[slot],
                kv_sems.at[slot, 1]).wait()
            if h + 1 < HQ:
                kv_fetch(h + 1, 1 - slot)

            qh = q_ref[:, h * DH:(h + 1) * DH]
            s = lax.dot_general(
                qh, kh_ref[slot], (((1,), (1,)), ((), ())),
                preferred_element_type=jnp.float32,
            ) * SCALE
            mh = jnp.max(s, axis=1, keepdims=True)
            p = jnp.exp(s - mh)
            lh = jnp.sum(p, axis=1, keepdims=True)
            oh = jnp.dot(p, vh_ref[slot], preferred_element_type=jnp.float32)
            for c in range(N_DEV):
                rows = slice(c * SQC, (c + 1) * SQC)
                loc_o[c, h] = oh[rows, :]
                loc_s[c, :, h:h + 1] = mh[rows, :]
                loc_s[c, :, HQ + h:HQ + h + 1] = lh[rows, :]

            if h % 2 == 1:
                hg = h // 2
                for d in range(1, N_DEV):
                    peer = (my + d) % N_DEV
                    rdma = pltpu.make_async_remote_copy(
                        src_ref=loc_o.at[peer, pl.ds(2 * hg, 2)],
                        dst_ref=rs_o.at[3 - d, pl.ds(2 * hg, 2)],
                        send_sem=rs_send_o.at[d - 1, hg],
                        recv_sem=rs_recv_o.at[3 - d, hg],
                        device_id=(peer,),
                        device_id_type=pl.DeviceIdType.MESH,
                    )
                    rdma.start()
                    sends.append(rdma)

        for d in range(1, N_DEV):
            peer = (my + d) % N_DEV
            rdma = pltpu.make_async_remote_copy(
                src_ref=loc_s.at[peer],
                dst_ref=rs_s.at[3 - d],
                send_sem=rs_send_s.at[d - 1],
                recv_sem=rs_recv_s.at[3 - d],
                device_id=(peer,),
                device_id_type=pl.DeviceIdType.MESH,
            )
            rdma.start()
            sends.append(rdma)
        cp_o = pltpu.make_async_copy(loc_o.at[my], my_o, self_sems.at[0])
        cp_s = pltpu.make_async_copy(loc_s.at[my], my_s, self_sems.at[1])
        cp_o.start()
        cp_s.start()
        cp_o.wait()
        cp_s.wait()

        for sl in range(N_DEV - 1):
            for hg in range(NHG):
                pltpu.make_async_remote_copy(
                    src_ref=rs_o.at[sl, pl.ds(2 * hg, 2)],
                    dst_ref=rs_o.at[sl, pl.ds(2 * hg, 2)],
                    send_sem=rs_send_o.at[0, 0],
                    recv_sem=rs_recv_o.at[sl, hg],
                    device_id=(my,),
                    device_id_type=pl.DeviceIdType.MESH,
                ).wait_recv()
            pltpu.make_async_remote_copy(
                src_ref=rs_s.at[sl],
                dst_ref=rs_s.at[sl],
                send_sem=rs_send_s.at[0],
                recv_sem=rs_recv_s.at[sl],
                device_id=(my,),
                device_id_type=pl.DeviceIdType.MESH,
            ).wait_recv()

        for h in range(HQ):
            ms = [my_s[:, h:h + 1]] + [
                rs_s[sl, :, h:h + 1] for sl in range(N_DEV - 1)]
            m_tot = jnp.maximum(jnp.maximum(ms[0], ms[1]),
                                jnp.maximum(ms[2], ms[3]))
            a0 = jnp.exp(ms[0] - m_tot)
            l_tot = my_s[:, HQ + h:HQ + h + 1] * a0
            o_tot = my_o[h] * a0
            for sl in range(N_DEV - 1):
                a = jnp.exp(ms[sl + 1] - m_tot)
                l_tot = l_tot + rs_s[sl, :, HQ + h:HQ + h + 1] * a
                o_tot = o_tot + rs_o[sl, h] * a
            attn_c[:, h * DH:(h + 1) * DH] = o_tot / l_tot

        rows_ref[...] = jnp.dot(attn_c[...], wo_ref[...],
                                preferred_element_type=jnp.float32)
        cp_rows = pltpu.make_async_copy(rows_ref, ag_out.at[my],
                                        self_sems.at[0])
        cp_rows.start()
        for d in range(1, N_DEV):
            peer = (my + d) % N_DEV
            rdma = pltpu.make_async_remote_copy(
                src_ref=rows_ref,
                dst_ref=ag_out.at[my],
                send_sem=ag_send.at[d - 1],
                recv_sem=ag_recv.at[3 - d],
                device_id=(peer,),
                device_id_type=pl.DeviceIdType.MESH,
            )
            rdma.start()
            sends.append(rdma)
        for sl in range(N_DEV - 1):
            pltpu.make_async_remote_copy(
                src_ref=rows_ref,
                dst_ref=ag_out.at[sl],
                send_sem=ag_send.at[0],
                recv_sem=ag_recv.at[sl],
                device_id=(my,),
                device_id_type=pl.DeviceIdType.MESH,
            ).wait_recv()
        cp_rows.wait()

        for c in range(N_DEV):
            out_ref[0, c * SQC:(c + 1) * SQC, :] = ag_out[c]

        for rdma in sends:
            rdma.wait_send()

    return pl.pallas_call(
        body,
        out_shape=jax.ShapeDtypeStruct((1, SQ, DM), jnp.float32),
        in_specs=[
            pl.BlockSpec(memory_space=pltpu.VMEM),
            pl.BlockSpec(memory_space=pltpu.VMEM),
            pl.BlockSpec(memory_space=pltpu.VMEM),
            pl.BlockSpec(memory_space=pl.ANY),
            pl.BlockSpec(memory_space=pl.ANY),
        ],
        out_specs=pl.BlockSpec(memory_space=pltpu.VMEM),
        scratch_shapes=[
            pltpu.VMEM((N_DEV, HQ, SQC, DH), jnp.float32),
            pltpu.VMEM((N_DEV, SQC, 2 * HQ), jnp.float32),
            pltpu.VMEM((N_DEV - 1, HQ, SQC, DH), jnp.float32),
            pltpu.VMEM((N_DEV - 1, SQC, 2 * HQ), jnp.float32),
            pltpu.VMEM((HQ, SQC, DH), jnp.float32),
            pltpu.VMEM((SQC, 2 * HQ), jnp.float32),
            pltpu.VMEM((SQ, DM), jnp.float32),
            pltpu.VMEM((SQC, DM), jnp.float32),
            pltpu.VMEM((SQC, DM), jnp.float32),
            pltpu.VMEM((N_DEV, SQC, DM), jnp.float32),
            pltpu.VMEM((2, SKV, DH), jnp.float32),
            pltpu.VMEM((2, SKV, DH), jnp.float32),
            pltpu.SemaphoreType.DMA((2, 2)),
            pltpu.SemaphoreType.DMA((2,)),
            pltpu.SemaphoreType.DMA((N_DEV - 1, NHG)),
            pltpu.SemaphoreType.DMA((N_DEV - 1, NHG)),
            pltpu.SemaphoreType.DMA((N_DEV - 1,)),
            pltpu.SemaphoreType.DMA((N_DEV - 1,)),
            pltpu.SemaphoreType.DMA((N_DEV - 1,)),
            pltpu.SemaphoreType.DMA((N_DEV - 1,)),
        ],
        compiler_params=pltpu.CompilerParams(
            collective_id=0,
            vmem_limit_bytes=100 * 1024 * 1024,
        ),
    )(x, Wq, Wo, K_ext, V_ext)
